# baseline (speedup 1.0000x reference)
import functools

import jax
import jax.numpy as jnp
from jax import lax
from jax.experimental import pallas as pl
from jax.experimental.pallas import tpu as pltpu
from jax.experimental.pallas import tpu_sc as plsc

B = 4096
L = 50
EMB = 300
VOCAB = 1000000
NC = 2
NS = 16
NW = NC * NS
BPW = B // NW
NRING = 4
BPG = 4
GROUPS = BPW // BPG
OFFS8 = tuple(16 * k for k in range(8))
EMBP = 304
C_LOAD_OFFS = (0, 16, 32)
C_STORE_OFFS = (256, 272, 288)


def _pool_body(idx_hbm, table_hbm, tail_hbm, out_hbm, idx_v, a0, a1, a2, a3,
               b0, b1, b2, b3, c0, c1, c2, c3, stage_v, sem0, sem1, sem2,
               sem3):
    c = lax.axis_index("c")
    s = lax.axis_index("s")
    wid = s * NC + c
    base = wid * BPW

    pltpu.sync_copy(idx_hbm.at[pl.ds(base, BPW)], idx_v)

    bufA = (a0, a1, a2, a3)
    bufB = (b0, b1, b2, b3)
    bufC = (c0, c1, c2, c3)
    sems = (sem0, sem1, sem2, sem3)

    def start(bi, p):
        il = idx_v.at[bi]
        pltpu.async_copy(table_hbm.at[il, pl.ds(0, 128)], bufA[p], sems[p])
        pltpu.async_copy(table_hbm.at[il, pl.ds(128, 128)], bufB[p], sems[p])
        pltpu.async_copy(tail_hbm.at[il], bufC[p], sems[p])

    def wait(bi, p):
        il = idx_v.at[bi]
        pltpu.make_async_copy(table_hbm.at[il, pl.ds(0, 128)], bufA[p],
                              sems[p]).wait()
        pltpu.make_async_copy(table_hbm.at[il, pl.ds(128, 128)], bufB[p],
                              sems[p]).wait()
        pltpu.make_async_copy(tail_hbm.at[il], bufC[p], sems[p]).wait()

    def accum(p, row):
        def rbody(l, accs):
            aA = tuple(x + bufA[p][l, pl.ds(off, 16)]
                       for x, off in zip(accs[0], OFFS8))
            aB = tuple(x + bufB[p][l, pl.ds(off, 16)]
                       for x, off in zip(accs[1], OFFS8))
            aC = tuple(x + bufC[p][l, pl.ds(off, 16)]
                       for x, off in zip(accs[2], C_LOAD_OFFS))
            return (aA, aB, aC)

        def zeros(n):
            return tuple(jnp.zeros((16,), jnp.float32) for _ in range(n))

        accs = lax.fori_loop(0, L, rbody, (zeros(8), zeros(8), zeros(3)))
        inv = jnp.float32(1.0 / L)
        for x, off in zip(accs[0], OFFS8):
            stage_v[row, pl.ds(off, 16)] = x * inv
        for x, off in zip(accs[1], OFFS8):
            stage_v[row, pl.ds(128 + off, 16)] = x * inv
        for x, off in zip(accs[2], C_STORE_OFFS):
            stage_v[row, pl.ds(off, 16)] = x * inv

    for pre in range(NRING - 1):
        start(pre, pre)

    def outer(grp, carry):
        for gb in range(BPG):
            p = gb % NRING
            bi = grp * BPG + gb
            nxt = bi + (NRING - 1)

            @pl.when(nxt < BPW)
            def _():
                start(nxt, (gb + NRING - 1) % NRING)

            wait(bi, p)
            accum(p, gb)
        pltpu.sync_copy(stage_v, out_hbm.at[pl.ds(base + grp * BPG, BPG)])
        return carry

    lax.fori_loop(0, GROUPS, outer, 0)


_pool = pl.kernel(
    _pool_body,
    out_type=jax.ShapeDtypeStruct((B, EMBP), jnp.float32),
    mesh=plsc.VectorSubcoreMesh(core_axis_name="c", subcore_axis_name="s"),
    scratch_types=[
        pltpu.VMEM((BPW, L), jnp.int32),
        pltpu.VMEM((L, 128), jnp.float32),
        pltpu.VMEM((L, 128), jnp.float32),
        pltpu.VMEM((L, 128), jnp.float32),
        pltpu.VMEM((L, 128), jnp.float32),
        pltpu.VMEM((L, 128), jnp.float32),
        pltpu.VMEM((L, 128), jnp.float32),
        pltpu.VMEM((L, 128), jnp.float32),
        pltpu.VMEM((L, 128), jnp.float32),
        pltpu.VMEM((L, 128), jnp.float32),
        pltpu.VMEM((L, 128), jnp.float32),
        pltpu.VMEM((L, 128), jnp.float32),
        pltpu.VMEM((L, 128), jnp.float32),
        pltpu.VMEM((BPG, EMBP), jnp.float32),
        pltpu.SemaphoreType.DMA,
        pltpu.SemaphoreType.DMA,
        pltpu.SemaphoreType.DMA,
        pltpu.SemaphoreType.DMA,
    ],
)

TAIL_R = 4000


def _tail_body(t_ref, o_ref):
    x = t_ref[...]
    o_ref[...] = jnp.concatenate(
        [x[:, 0:44], jnp.zeros((x.shape[0], 84), jnp.float32)], axis=1)


_tail = pl.pallas_call(
    _tail_body,
    grid=(VOCAB // TAIL_R,),
    in_specs=[pl.BlockSpec((TAIL_R, 128), lambda i: (i, 2))],
    out_specs=pl.BlockSpec((TAIL_R, 128), lambda i: (i, 0)),
    out_shape=jax.ShapeDtypeStruct((VOCAB, 128), jnp.float32),
)


def _mlp_body(x_ref, w1_ref, b1_ref, w2_ref, b2_ref, w3_ref, b3_ref, w4_ref,
              b4_ref, o_ref):
    x = x_ref[...]
    h = jnp.maximum(
        jnp.dot(x, w1_ref[...], preferred_element_type=jnp.float32) +
        b1_ref[...], 0.0)
    h = jnp.maximum(
        jnp.dot(h, w2_ref[...], preferred_element_type=jnp.float32) +
        b2_ref[...], 0.0)
    h = jnp.maximum(
        jnp.dot(h, w3_ref[...], preferred_element_type=jnp.float32) +
        b3_ref[...], 0.0)
    z = jnp.dot(h, w4_ref[...], preferred_element_type=jnp.float32) + \
        b4_ref[...]
    o_ref[...] = jax.nn.sigmoid(z)


_mlp = pl.pallas_call(
    _mlp_body,
    out_shape=jax.ShapeDtypeStruct((B, 1), jnp.float32),
)


@jax.jit
def kernel(inputs, table, W1, b1, W2, b2, W3, b3, W4, b4):
    idx = inputs.astype(jnp.int32)
    tail = _tail(table)
    pooled = _pool(idx, table, tail)
    w1p = jnp.pad(W1, ((0, EMBP - EMB), (0, 0)))
    return _mlp(pooled, w1p, b1.reshape(1, -1), W2, b2.reshape(1, -1), W3,
                b3.reshape(1, -1), W4, b4.reshape(1, -1))
